# Initial kernel scaffold; baseline (speedup 1.0000x reference)
import jax
import jax.numpy as jnp
from jax.experimental import pallas as pl


def kernel(pred_labels, class_probits, pred_boxes, gt_labels, gt_boxes):
    raise NotImplementedError("write your pallas kernel here")


# speedup vs baseline: 2495.0822x; 2495.0822x over previous
import jax
import jax.numpy as jnp
from jax.experimental import pallas as pl

_EPS = 1e-05
_IOU_THR = 0.5
_NP = 5120
_NG = 512
_NEG = -jnp.inf


def _map_body(pred_ref, gt_ref, tpts_ref, out_ref):
    p = pred_ref[...]
    g = gt_ref[...]
    tpts = tpts_ref[...]

    px1 = p[0:1, :]; py1 = p[1:2, :]; px2 = p[2:3, :]; py2 = p[3:4, :]
    prob = p[4:5, :]; plab = p[5:6, :]
    gx1 = g[:, 0:1]; gy1 = g[:, 1:2]; gx2 = g[:, 2:3]; gy2 = g[:, 3:4]
    glab = g[:, 4:5]

    area_p = (px2 - px1) * (py2 - py1)
    area_g = (gx2 - gx1) * (gy2 - gy1)
    ltx = jnp.maximum(gx1, px1)
    lty = jnp.maximum(gy1, py1)
    rbx = jnp.minimum(gx2, px2)
    rby = jnp.minimum(gy2, py2)
    w = jnp.maximum(rbx - ltx, 0.0)
    h = jnp.maximum(rby - lty, 0.0)
    inter = w * h
    iou = inter / (area_g + area_p - inter + 1e-12)

    rowid = jax.lax.broadcasted_iota(jnp.int32, (_NG, _NP), 0)
    colid = jax.lax.broadcasted_iota(jnp.int32, (1, _NP), 1)
    di = jax.lax.broadcasted_iota(jnp.int32, (_NG, _NG), 0)
    dj = jax.lax.broadcasted_iota(jnp.int32, (_NG, _NG), 1)
    diag = di == dj

    total = jnp.float32(0.0)
    for c in (1.0, 2.0, 3.0):
        gm = glab == c
        iou_c = jnp.where(gm, iou, 0.0)
        maxv = jnp.max(iou_c, axis=0, keepdims=True)
        chosen = jnp.min(jnp.where(iou_c == maxv, rowid, _NG),
                         axis=0, keepdims=True)
        member = plab == c
        cand = member & (maxv > _IOU_THR)
        sel = cand & (chosen == rowid)
        m1 = jnp.max(jnp.where(sel, prob, _NEG), axis=1, keepdims=True)
        exists = m1 > _NEG
        widx = jnp.min(jnp.where(sel & (prob == m1), colid, _NP),
                       axis=1, keepdims=True)
        beats = member & ((prob > m1) | ((prob == m1) & (colid < widx)))
        r = jnp.sum(beats.astype(jnp.float32), axis=1, keepdims=True)
        widxf = widx.astype(jnp.float32)
        m1t = jnp.max(jnp.where(diag, m1, _NEG), axis=0, keepdims=True)
        widxt = jnp.min(jnp.where(diag, widxf, jnp.float32(_NP + 1)),
                        axis=0, keepdims=True)
        betterw = (m1t > m1) | ((m1t == m1) & (widxt < widxf))
        k = 1.0 + jnp.sum(betterw.astype(jnp.float32), axis=1, keepdims=True)
        num_gt = jnp.sum(gm.astype(jnp.float32))
        prec = k / (r + 1.0 + _EPS)
        recall = k / (num_gt + _EPS)
        elig = exists & (recall >= tpts)
        pmax = jnp.max(jnp.where(elig, prec, _NEG), axis=0, keepdims=True)
        any_e = jnp.max(elig.astype(jnp.float32), axis=0, keepdims=True) > 0
        ap = jnp.sum(jnp.where(any_e, pmax, 0.0)) / 11.0
        nmem = jnp.sum(member.astype(jnp.float32))
        valid = (nmem > 0) & (num_gt > 0)
        total = total + jnp.where(valid, ap, 0.0)

    out_ref[...] = jnp.broadcast_to(total / 3.0, (1, 128))


def kernel(pred_labels, class_probits, pred_boxes, gt_labels, gt_boxes):
    np0 = pred_boxes.shape[0]
    ng0 = gt_boxes.shape[0]
    pred = jnp.zeros((8, _NP), jnp.float32)
    pred = pred.at[0:4, :np0].set(pred_boxes.T.astype(jnp.float32))
    pred = pred.at[4, :np0].set(class_probits.astype(jnp.float32))
    pred = pred.at[5, :np0].set(pred_labels.astype(jnp.float32))
    pred = pred.at[5, np0:].set(-1.0)
    gt = jnp.zeros((_NG, 8), jnp.float32)
    gt = gt.at[:ng0, 0:4].set(gt_boxes.astype(jnp.float32))
    gt = gt.at[:ng0, 4].set(gt_labels.astype(jnp.float32))
    gt = gt.at[ng0:, 4].set(-1.0)
    tpts = jnp.full((1, 16), 2.0, jnp.float32)
    tpts = tpts.at[0, :11].set(jnp.arange(0.0, 1.1, 0.1, dtype=jnp.float32))
    out = pl.pallas_call(
        _map_body,
        out_shape=jax.ShapeDtypeStruct((1, 128), jnp.float32),
    )(pred, gt, tpts)
    return out[0, 0]
